# baseline (speedup 1.0000x reference)
import jax
import jax.numpy as jnp
from jax import lax
from jax.experimental import pallas as pl
from jax.experimental.pallas import tpu as pltpu
from jax.experimental.pallas import tpu_sc as plsc

B = 16384
NCAT = 26
NCONT = 26
NCOLS = 52
V = 100000
D = 32
OUT_W = NCONT + NCAT * D
XP = 128

NC = 2
NS = 16
NW = NC * NS
RW = B // NW
M = 16
NG = RW // M
NROW = M * NCAT
GROUP = 104
NGRP = NROW // GROUP

QA = 25088
TK = 7
TV = QA // TK
TAIL = 3200
LAST = 32
TROWS = NCAT * QA
TSTEPS = NCAT * TK


def _transpose_body(tab_ref, last_ref, out_ref, buf, sems):
    c = pl.program_id(0)
    k = pl.program_id(1)
    i = c * TK + k
    slot = lax.rem(i, 2)
    nxt = lax.rem(i + 1, 2)

    def start_copies(cc, kk, sl):
        for r in range(3):
            pltpu.make_async_copy(
                tab_ref.at[cc, :, pl.ds(r * QA + kk * TV, TV)],
                buf.at[sl, r], sems.at[sl, r]).start()

        @pl.when(kk < TK - 1)
        def _full():
            pltpu.make_async_copy(
                tab_ref.at[cc, :, pl.ds(3 * QA + kk * TV, TV)],
                buf.at[sl, 3], sems.at[sl, 3]).start()

        @pl.when(kk == TK - 1)
        def _tail():
            pltpu.make_async_copy(
                tab_ref.at[cc, :, pl.ds(3 * QA + kk * TV, TAIL)],
                buf.at[sl, 3].at[:, pl.ds(0, TAIL)], sems.at[sl, 3]).start()

    def wait_copies(cc, kk, sl):
        for r in range(3):
            pltpu.make_async_copy(
                tab_ref.at[cc, :, pl.ds(r * QA + kk * TV, TV)],
                buf.at[sl, r], sems.at[sl, r]).wait()

        @pl.when(kk < TK - 1)
        def _full():
            pltpu.make_async_copy(
                tab_ref.at[cc, :, pl.ds(3 * QA + kk * TV, TV)],
                buf.at[sl, 3], sems.at[sl, 3]).wait()

        @pl.when(kk == TK - 1)
        def _tail():
            pltpu.make_async_copy(
                tab_ref.at[cc, :, pl.ds(3 * QA + kk * TV, TAIL)],
                buf.at[sl, 3].at[:, pl.ds(0, TAIL)], sems.at[sl, 3]).wait()

    @pl.when(i == 0)
    def _first():
        start_copies(c, k, slot)

    @pl.when(i < TSTEPS - 1)
    def _prefetch():
        start_copies(lax.div(i + 1, TK), lax.rem(i + 1, TK), nxt)

    wait_copies(c, k, slot)

    @pl.when(k == TK - 1)
    def _merge_tail():
        buf[slot, 3, :, pl.ds(TAIL, LAST)] = last_ref[c]

    eye = (lax.broadcasted_iota(jnp.int32, (D, D), 0)
           == lax.broadcasted_iota(jnp.int32, (D, D), 1)).astype(jnp.float32)
    for r in range(4):
        t = lax.dot_general(
            buf[slot, r], eye,
            dimension_numbers=(((0,), (0,)), ((), ())),
            preferred_element_type=jnp.float32,
            precision=lax.Precision.HIGHEST)
        out_ref[:, r * D:(r + 1) * D] = t


def _gather_body(x_hbm, tab_hbm, emb_hbm, x_v, idx_v, sel_v, rows_v, stage_v,
                 sem):
    wid = lax.axis_index("s") * NC + lax.axis_index("c")
    iota = lax.iota(jnp.int32, 16)
    pat_a = iota * QA
    pat_b = (iota + 10) * QA

    @pl.loop(0, NG)
    def _chunk(g):
        base = wid * RW + g * M

        pltpu.sync_copy(x_hbm.at[pl.ds(base, M)], x_v)

        @pl.loop(0, M)
        def _build(b):
            va = x_v[b, pl.ds(NCONT, 16)]
            vb = x_v[b, pl.ds(NCONT + 10, 16)]
            ra = ((va >= QA).astype(jnp.int32) + (va >= 2 * QA)
                  + (va >= 3 * QA))
            rb = ((vb >= QA).astype(jnp.int32) + (vb >= 2 * QA)
                  + (vb >= 3 * QA))
            idx_v[pl.ds(b * NCAT, 16)] = pat_a + va - ra * QA
            idx_v[pl.ds(b * NCAT + 10, 16)] = pat_b + vb - rb * QA
            sel_v[pl.ds(b * NCAT, 16)] = ra * D
            sel_v[pl.ds(b * NCAT + 10, 16)] = rb * D

        @pl.loop(0, NGRP)
        def _fire(j):
            pltpu.async_copy(
                tab_hbm.at[idx_v.at[pl.ds(j * GROUP, GROUP)]],
                rows_v.at[pl.ds(j * GROUP, GROUP)],
                sem,
            )

        pltpu.make_async_copy(tab_hbm.at[pl.ds(0, NROW)], rows_v, sem).wait()

        @pl.loop(0, M)
        def _extract(b):
            brow = jnp.full((16,), 0, jnp.int32) + b
            for coff in (0, 10):
                p = b * NCAT + coff + iota
                off = sel_v[pl.ds(b * NCAT + coff, 16)]
                col0 = (iota + coff) * D
                for w in range(D):
                    vals = plsc.load_gather(rows_v, [p, off + w])
                    plsc.store_scatter(stage_v, [brow, col0 + w], vals)

        pltpu.sync_copy(stage_v, emb_hbm.at[pl.ds(base, M)])


def _interleave_body(x_ref, emb_ref, out_ref):
    cont = x_ref[:, :NCONT].astype(jnp.float32)
    blk = jnp.concatenate([cont, emb_ref[...]], axis=1)
    out_ref[...] = blk.T


BM = 512


@jax.jit
def _run(x, tab_t):
    tab128 = pl.pallas_call(
        _transpose_body,
        out_shape=jax.ShapeDtypeStruct((TROWS, 128), jnp.float32),
        grid=(NCAT, TK),
        in_specs=[
            pl.BlockSpec(memory_space=pltpu.MemorySpace.HBM),
            pl.BlockSpec((NCAT, D, LAST), lambda c, k: (0, 0, 0)),
        ],
        out_specs=pl.BlockSpec((TV, 128), lambda c, k: (c * TK + k, 0)),
        scratch_shapes=[
            pltpu.VMEM((2, 4, D, TV), jnp.float32),
            pltpu.SemaphoreType.DMA((2, 4)),
        ],
    )(tab_t, tab_t[:, :, V - LAST:V])

    xp = jnp.pad(x, ((0, 0), (0, XP - NCOLS)))
    gather = pl.kernel(
        _gather_body,
        out_type=jax.ShapeDtypeStruct((B, NCAT * D), jnp.float32),
        mesh=plsc.VectorSubcoreMesh(core_axis_name="c", subcore_axis_name="s"),
        compiler_params=pltpu.CompilerParams(use_tc_tiling_on_sc=True, needs_layout_passes=False),
        scratch_types=[
            pltpu.VMEM((M, XP), jnp.int32),
            pltpu.VMEM((NROW,), jnp.int32),
            pltpu.VMEM((NROW,), jnp.int32),
            pltpu.VMEM((NROW, 128), jnp.float32),
            pltpu.VMEM((M, NCAT * D), jnp.float32),
            pltpu.SemaphoreType.DMA,
        ],
    )
    emb = gather(xp, tab128)

    out_t = pl.pallas_call(
        _interleave_body,
        out_shape=jax.ShapeDtypeStruct((OUT_W, B), jnp.float32),
        grid=(B // BM,),
        in_specs=[
            pl.BlockSpec((BM, NCOLS), lambda i: (i, 0)),
            pl.BlockSpec((BM, NCAT * D), lambda i: (i, 0)),
        ],
        out_specs=pl.BlockSpec((OUT_W, BM), lambda i: (0, i)),
    )(x, emb)
    return out_t.T


def kernel(x, tables):
    return _run(x, tables.transpose(0, 2, 1))


# speedup vs baseline: 2.8737x; 2.8737x over previous
import jax
import jax.numpy as jnp
from jax import lax
from jax.experimental import pallas as pl
from jax.experimental.pallas import tpu as pltpu
from jax.experimental.pallas import tpu_sc as plsc

B = 16384
NCAT = 26
NCONT = 26
NCOLS = 52
V = 100000
D = 32
OUT_W = NCONT + NCAT * D
XP = 128

NC = 2
NS = 16
NW = NC * NS
RW = B // NW
M = 16
NG = RW // M
NROW = M * NCAT
GROUP = 104
NGRP = NROW // GROUP

QA = 25088
TK = 7
TV = QA // TK
TAIL = 3200
LAST = 32
TROWS = NCAT * QA
TSTEPS = NCAT * TK


def _transpose_body(tab_ref, last_ref, out_ref, buf, sems):
    c = pl.program_id(0)
    k = pl.program_id(1)
    i = c * TK + k
    slot = lax.rem(i, 2)
    nxt = lax.rem(i + 1, 2)

    def copy_descs(cc, kk, sl, tail):
        descs = []
        for r in range(3):
            descs.append(pltpu.make_async_copy(
                tab_ref.at[cc, :, pl.ds(r * QA + kk * TV, TV)],
                buf.at[sl, pl.ds(r * D, D)], sems.at[sl, r]))
        if tail:
            descs.append(pltpu.make_async_copy(
                tab_ref.at[cc, :, pl.ds(3 * QA + kk * TV, TAIL)],
                buf.at[sl, pl.ds(3 * D, D)].at[:, pl.ds(0, TAIL)],
                sems.at[sl, 3]))
        else:
            descs.append(pltpu.make_async_copy(
                tab_ref.at[cc, :, pl.ds(3 * QA + kk * TV, TV)],
                buf.at[sl, pl.ds(3 * D, D)], sems.at[sl, 3]))
        return descs

    def start_copies(cc, kk, sl):
        @pl.when(kk < TK - 1)
        def _full():
            for d in copy_descs(cc, kk, sl, False):
                d.start()

        @pl.when(kk == TK - 1)
        def _tail():
            for d in copy_descs(cc, kk, sl, True):
                d.start()

    def wait_copies(cc, kk, sl):
        @pl.when(kk < TK - 1)
        def _full():
            for d in copy_descs(cc, kk, sl, False):
                d.wait()

        @pl.when(kk == TK - 1)
        def _tail():
            for d in copy_descs(cc, kk, sl, True):
                d.wait()

    @pl.when(i == 0)
    def _first():
        start_copies(c, k, slot)

    @pl.when(i < TSTEPS - 1)
    def _prefetch():
        start_copies(lax.div(i + 1, TK), lax.rem(i + 1, TK), nxt)

    wait_copies(c, k, slot)

    @pl.when(k == TK - 1)
    def _merge_tail():
        buf[slot, pl.ds(3 * D, D), pl.ds(TAIL, LAST)] = last_ref[c]

    eye = (lax.broadcasted_iota(jnp.int32, (128, 128), 0)
           == lax.broadcasted_iota(jnp.int32, (128, 128), 1)
           ).astype(jnp.float32)
    out_ref[...] = lax.dot_general(
        buf[slot], eye,
        dimension_numbers=(((0,), (0,)), ((), ())),
        preferred_element_type=jnp.float32,
        precision=lax.Precision.HIGHEST)


def _take16(vec, lane):
    dnums = lax.GatherDimensionNumbers(
        offset_dims=(), collapsed_slice_dims=(0,), start_index_map=(0,))
    idx = jnp.full((16, 1), lane, jnp.int32)
    return lax.gather(vec, idx, dnums, (1,),
                      mode=lax.GatherScatterMode.PROMISE_IN_BOUNDS)


def _gather_body(x_hbm, tab_hbm, emb_hbm, x_v, idx_v, sel_v, rows_v,
                 stage_v, sem):
    wid = lax.axis_index("s") * NC + lax.axis_index("c")
    iota = lax.iota(jnp.int32, 16)
    pat_a = iota * QA
    pat_b = (iota + 10) * QA

    @pl.loop(0, NG)
    def _chunk(g):
        base = wid * RW + g * M

        pltpu.sync_copy(x_hbm.at[pl.ds(base, M)], x_v)

        @pl.loop(0, M)
        def _build(b):
            va = x_v[b, pl.ds(NCONT, 16)]
            vb = x_v[b, pl.ds(NCONT + 10, 16)]
            ra = ((va >= QA).astype(jnp.int32) + (va >= 2 * QA)
                  + (va >= 3 * QA))
            rb = ((vb >= QA).astype(jnp.int32) + (vb >= 2 * QA)
                  + (vb >= 3 * QA))
            idx_v[pl.ds(b * NCAT, 16)] = pat_a + va - ra * QA
            idx_v[pl.ds(b * NCAT + 10, 16)] = pat_b + vb - rb * QA
            sel_v[pl.ds(b * NCAT, 16)] = ra * D
            sel_v[pl.ds(b * NCAT + 10, 16)] = rb * D

        @pl.loop(0, NGRP)
        def _fire(j):
            pltpu.async_copy(
                tab_hbm.at[idx_v.at[pl.ds(j * GROUP, GROUP)]],
                rows_v.at[pl.ds(j * GROUP, GROUP)],
                sem,
            )

        pltpu.make_async_copy(tab_hbm.at[pl.ds(0, NROW)], rows_v, sem).wait()

        @pl.loop(0, M)
        def _extract(b):
            ga = sel_v[pl.ds(b * NCAT, 16)]
            gb = sel_v[pl.ds(b * NCAT + 10, 16)]
            for ci in range(NCAT):
                p = b * NCAT + ci
                grp, lane = (ga, ci) if ci < 10 else (gb, ci - 10)
                off = _take16(grp, lane)
                for half in range(2):
                    w = half * 16
                    r0 = rows_v[p, pl.ds(0 + w, 16)]
                    r1 = rows_v[p, pl.ds(32 + w, 16)]
                    r2 = rows_v[p, pl.ds(64 + w, 16)]
                    r3 = rows_v[p, pl.ds(96 + w, 16)]
                    val = jnp.where(
                        off == 0, r0,
                        jnp.where(off == D, r1,
                                  jnp.where(off == 2 * D, r2, r3)))
                    stage_v[b, pl.ds(ci * D + w, 16)] = val

        pltpu.sync_copy(stage_v, emb_hbm.at[pl.ds(base, M)])


def _interleave_body(x_ref, emb_ref, out_ref):
    cont = x_ref[:, :NCONT].astype(jnp.float32)
    blk = jnp.concatenate([cont, emb_ref[...]], axis=1)
    out_ref[...] = blk.T


BM = 512


@jax.jit
def _run(x, tab_t):
    tab128 = pl.pallas_call(
        _transpose_body,
        out_shape=jax.ShapeDtypeStruct((TROWS, 128), jnp.float32),
        grid=(NCAT, TK),
        in_specs=[
            pl.BlockSpec(memory_space=pltpu.MemorySpace.HBM),
            pl.BlockSpec((NCAT, D, LAST), lambda c, k: (0, 0, 0)),
        ],
        out_specs=pl.BlockSpec((TV, 128), lambda c, k: (c * TK + k, 0)),
        scratch_shapes=[
            pltpu.VMEM((2, 128, TV), jnp.float32),
            pltpu.SemaphoreType.DMA((2, 4)),
        ],
    )(tab_t, tab_t[:, :, V - LAST:V])

    xp = jnp.pad(x, ((0, 0), (0, XP - NCOLS)))
    gather = pl.kernel(
        _gather_body,
        out_type=jax.ShapeDtypeStruct((B, NCAT * D), jnp.float32),
        mesh=plsc.VectorSubcoreMesh(core_axis_name="c", subcore_axis_name="s"),
        compiler_params=pltpu.CompilerParams(use_tc_tiling_on_sc=True, needs_layout_passes=False),
        scratch_types=[
            pltpu.VMEM((M, XP), jnp.int32),
            pltpu.VMEM((NROW,), jnp.int32),
            pltpu.VMEM((NROW,), jnp.int32),
            pltpu.VMEM((NROW, 128), jnp.float32),
            pltpu.VMEM((M, NCAT * D), jnp.float32),
            pltpu.SemaphoreType.DMA,
        ],
    )
    emb = gather(xp, tab128)

    out_t = pl.pallas_call(
        _interleave_body,
        out_shape=jax.ShapeDtypeStruct((OUT_W, B), jnp.float32),
        grid=(B // BM,),
        in_specs=[
            pl.BlockSpec((BM, NCOLS), lambda i: (i, 0)),
            pl.BlockSpec((BM, NCAT * D), lambda i: (i, 0)),
        ],
        out_specs=pl.BlockSpec((OUT_W, BM), lambda i: (0, i)),
    )(x, emb)
    return out_t.T


def kernel(x, tables):
    return _run(x, tables.transpose(0, 2, 1))


# speedup vs baseline: 3.6249x; 1.2614x over previous
import jax
import jax.numpy as jnp
from jax import lax
from jax.experimental import pallas as pl
from jax.experimental.pallas import tpu as pltpu
from jax.experimental.pallas import tpu_sc as plsc

B = 16384
NCAT = 26
NCONT = 26
NCOLS = 52
V = 100000
D = 32
OUT_W = NCONT + NCAT * D
XP = 128

NC = 2
NS = 16
NW = NC * NS
RW = B // NW
M = 32
NG = RW // M
NROW = M * NCAT
GROUP = 104
NGRP = NROW // GROUP

QA = 25088
TK = 7
TV = QA // TK
TAIL = 3200
LAST = 32
TROWS = NCAT * QA
TSTEPS = NCAT * TK


def _transpose_body(tab_ref, last_ref, out_ref, buf, sems):
    c = pl.program_id(0)
    k = pl.program_id(1)
    i = c * TK + k
    slot = lax.rem(i, 2)
    nxt = lax.rem(i + 1, 2)

    def copy_descs(cc, kk, sl, tail):
        descs = []
        for r in range(3):
            descs.append(pltpu.make_async_copy(
                tab_ref.at[cc, :, pl.ds(r * QA + kk * TV, TV)],
                buf.at[sl, pl.ds(r * D, D)], sems.at[sl, r]))
        if tail:
            descs.append(pltpu.make_async_copy(
                tab_ref.at[cc, :, pl.ds(3 * QA + kk * TV, TAIL)],
                buf.at[sl, pl.ds(3 * D, D)].at[:, pl.ds(0, TAIL)],
                sems.at[sl, 3]))
        else:
            descs.append(pltpu.make_async_copy(
                tab_ref.at[cc, :, pl.ds(3 * QA + kk * TV, TV)],
                buf.at[sl, pl.ds(3 * D, D)], sems.at[sl, 3]))
        return descs

    def start_copies(cc, kk, sl):
        @pl.when(kk < TK - 1)
        def _full():
            for d in copy_descs(cc, kk, sl, False):
                d.start()

        @pl.when(kk == TK - 1)
        def _tail():
            for d in copy_descs(cc, kk, sl, True):
                d.start()

    def wait_copies(cc, kk, sl):
        @pl.when(kk < TK - 1)
        def _full():
            for d in copy_descs(cc, kk, sl, False):
                d.wait()

        @pl.when(kk == TK - 1)
        def _tail():
            for d in copy_descs(cc, kk, sl, True):
                d.wait()

    @pl.when(i == 0)
    def _first():
        start_copies(c, k, slot)

    @pl.when(i < TSTEPS - 1)
    def _prefetch():
        start_copies(lax.div(i + 1, TK), lax.rem(i + 1, TK), nxt)

    wait_copies(c, k, slot)

    @pl.when(k == TK - 1)
    def _merge_tail():
        buf[slot, pl.ds(3 * D, D), pl.ds(TAIL, LAST)] = last_ref[c]

    eye = (lax.broadcasted_iota(jnp.int32, (128, 128), 0)
           == lax.broadcasted_iota(jnp.int32, (128, 128), 1)
           ).astype(jnp.float32)
    out_ref[...] = lax.dot_general(
        buf[slot], eye,
        dimension_numbers=(((0,), (0,)), ((), ())),
        preferred_element_type=jnp.float32,
        precision=lax.Precision.HIGHEST)


def _take16(vec, lane):
    dnums = lax.GatherDimensionNumbers(
        offset_dims=(), collapsed_slice_dims=(0,), start_index_map=(0,))
    idx = jnp.full((16, 1), lane, jnp.int32)
    return lax.gather(vec, idx, dnums, (1,),
                      mode=lax.GatherScatterMode.PROMISE_IN_BOUNDS)


def _gather_body(x_hbm, tab_hbm, emb_hbm, x_v, idx_v, rows_v,
                 stage_v, sem):
    wid = lax.axis_index("s") * NC + lax.axis_index("c")
    iota = lax.iota(jnp.int32, 16)
    pat_a = iota * QA
    pat_b = (iota + 10) * QA

    @pl.loop(0, NG)
    def _chunk(g):
        base = wid * RW + g * M

        pltpu.sync_copy(x_hbm.at[pl.ds(base, M)], x_v)

        @pl.loop(0, M)
        def _build(b):
            va = x_v[b, pl.ds(NCONT, 16)]
            vb = x_v[b, pl.ds(NCONT + 10, 16)]
            ra = ((va >= QA).astype(jnp.int32) + (va >= 2 * QA)
                  + (va >= 3 * QA))
            rb = ((vb >= QA).astype(jnp.int32) + (vb >= 2 * QA)
                  + (vb >= 3 * QA))
            idx_v[pl.ds(b * NCAT, 16)] = (pat_a + va - ra * QA) * 4 + ra
            idx_v[pl.ds(b * NCAT + 10, 16)] = (pat_b + vb - rb * QA) * 4 + rb

        @pl.loop(0, NGRP)
        def _fire(j):
            pltpu.async_copy(
                tab_hbm.at[idx_v.at[pl.ds(j * GROUP, GROUP)]],
                rows_v.at[pl.ds(j * GROUP, GROUP)],
                sem,
            )

        pltpu.make_async_copy(tab_hbm.at[pl.ds(0, NROW)], rows_v, sem).wait()

        @pl.loop(0, M)
        def _extract(b):
            for ci in range(NCAT):
                p = b * NCAT + ci
                stage_v[b, pl.ds(ci * D, 16)] = rows_v[p, pl.ds(0, 16)]
                stage_v[b, pl.ds(ci * D + 16, 16)] = rows_v[p, pl.ds(16, 16)]

        pltpu.sync_copy(stage_v, emb_hbm.at[pl.ds(base, M)])


def _interleave_body(x_ref, emb_ref, out_ref):
    cont = x_ref[:, :NCONT].astype(jnp.float32)
    blk = jnp.concatenate([cont, emb_ref[...]], axis=1)
    out_ref[...] = blk.T


BM = 512


@jax.jit
def _run(x, tab_t):
    tab128 = pl.pallas_call(
        _transpose_body,
        out_shape=jax.ShapeDtypeStruct((TROWS, 128), jnp.float32),
        grid=(NCAT, TK),
        in_specs=[
            pl.BlockSpec(memory_space=pltpu.MemorySpace.HBM),
            pl.BlockSpec((NCAT, D, LAST), lambda c, k: (0, 0, 0)),
        ],
        out_specs=pl.BlockSpec((TV, 128), lambda c, k: (c * TK + k, 0)),
        scratch_shapes=[
            pltpu.VMEM((2, 128, TV), jnp.float32),
            pltpu.SemaphoreType.DMA((2, 4)),
        ],
    )(tab_t, tab_t[:, :, V - LAST:V])

    xp = jnp.pad(x, ((0, 0), (0, XP - NCOLS)))
    gather = pl.kernel(
        _gather_body,
        out_type=jax.ShapeDtypeStruct((B, NCAT * D), jnp.float32),
        mesh=plsc.VectorSubcoreMesh(core_axis_name="c", subcore_axis_name="s"),
        compiler_params=pltpu.CompilerParams(use_tc_tiling_on_sc=False,
                                             needs_layout_passes=False),
        scratch_types=[
            pltpu.VMEM((M, XP), jnp.int32),
            pltpu.VMEM((NROW,), jnp.int32),
            pltpu.VMEM((NROW, D), jnp.float32),
            pltpu.VMEM((M, NCAT * D), jnp.float32),
            pltpu.SemaphoreType.DMA,
        ],
    )
    emb = gather(xp, tab128.reshape(4 * TROWS, D))

    out_t = pl.pallas_call(
        _interleave_body,
        out_shape=jax.ShapeDtypeStruct((OUT_W, B), jnp.float32),
        grid=(B // BM,),
        in_specs=[
            pl.BlockSpec((BM, NCOLS), lambda i: (i, 0)),
            pl.BlockSpec((BM, NCAT * D), lambda i: (i, 0)),
        ],
        out_specs=pl.BlockSpec((OUT_W, BM), lambda i: (0, i)),
    )(x, emb)
    return out_t.T


def kernel(x, tables):
    return _run(x, tables.transpose(0, 2, 1))


# speedup vs baseline: 3.8279x; 1.0560x over previous
import functools

import jax
import jax.numpy as jnp
from jax import lax
from jax.experimental import pallas as pl
from jax.experimental.pallas import tpu as pltpu
from jax.experimental.pallas import tpu_sc as plsc

B = 16384
NCAT = 26
NCONT = 26
NCOLS = 52
V = 100000
D = 32
OUT_W = NCONT + NCAT * D
XP = 128
HC = NCAT // 2

NC = 2
NS = 16
NW = NC * NS
RW = B // NW
M = 32
NG = RW // M
NROW = M * HC
GROUP = 104
NGRP = NROW // GROUP

QA = 25088
TK = 7
TV = QA // TK
TAIL = 3200
LAST = 32
HROWS = HC * QA
TSTEPS = HC * TK


def _transpose_body(c0, tab_ref, last_ref, out_ref, buf, sems):
    c = pl.program_id(0)
    k = pl.program_id(1)
    i = c * TK + k
    slot = lax.rem(i, 2)
    nxt = lax.rem(i + 1, 2)

    def copy_descs(cc, kk, sl, tail):
        descs = []
        for r in range(3):
            descs.append(pltpu.make_async_copy(
                tab_ref.at[cc + c0, :, pl.ds(r * QA + kk * TV, TV)],
                buf.at[sl, pl.ds(r * D, D)], sems.at[sl, r]))
        if tail:
            descs.append(pltpu.make_async_copy(
                tab_ref.at[cc + c0, :, pl.ds(3 * QA + kk * TV, TAIL)],
                buf.at[sl, pl.ds(3 * D, D)].at[:, pl.ds(0, TAIL)],
                sems.at[sl, 3]))
        else:
            descs.append(pltpu.make_async_copy(
                tab_ref.at[cc + c0, :, pl.ds(3 * QA + kk * TV, TV)],
                buf.at[sl, pl.ds(3 * D, D)], sems.at[sl, 3]))
        return descs

    def start_copies(cc, kk, sl):
        @pl.when(kk < TK - 1)
        def _full():
            for d in copy_descs(cc, kk, sl, False):
                d.start()

        @pl.when(kk == TK - 1)
        def _tail():
            for d in copy_descs(cc, kk, sl, True):
                d.start()

    def wait_copies(cc, kk, sl):
        @pl.when(kk < TK - 1)
        def _full():
            for d in copy_descs(cc, kk, sl, False):
                d.wait()

        @pl.when(kk == TK - 1)
        def _tail():
            for d in copy_descs(cc, kk, sl, True):
                d.wait()

    @pl.when(i == 0)
    def _first():
        start_copies(c, k, slot)

    @pl.when(i < TSTEPS - 1)
    def _prefetch():
        start_copies(lax.div(i + 1, TK), lax.rem(i + 1, TK), nxt)

    wait_copies(c, k, slot)

    @pl.when(k == TK - 1)
    def _merge_tail():
        buf[slot, pl.ds(3 * D, D), pl.ds(TAIL, LAST)] = last_ref[c + c0]

    eye = (lax.broadcasted_iota(jnp.int32, (128, 128), 0)
           == lax.broadcasted_iota(jnp.int32, (128, 128), 1)
           ).astype(jnp.float32)
    out_ref[...] = lax.dot_general(
        buf[slot], eye,
        dimension_numbers=(((0,), (0,)), ((), ())),
        preferred_element_type=jnp.float32,
        precision=lax.Precision.HIGHEST)


def _gather_body(c0, x_hbm, tab_hbm, emb_hbm, x_v, idx_v, rows_v, stage_v,
                 sem):
    wid = lax.axis_index("s") * NC + lax.axis_index("c")
    iota = lax.iota(jnp.int32, 16)
    pat = iota * QA

    @pl.loop(0, NG)
    def _chunk(g):
        base = wid * RW + g * M

        pltpu.sync_copy(x_hbm.at[pl.ds(base, M)], x_v)

        @pl.loop(0, M)
        def _build(b):
            va = x_v[b, pl.ds(NCONT + c0, 16)]
            ra = ((va >= QA).astype(jnp.int32) + (va >= 2 * QA)
                  + (va >= 3 * QA))
            idx = (pat + va - ra * QA) * 4 + ra
            idx_v[pl.ds(b * HC, 16)] = jnp.where(iota < HC, idx, 0)

        @pl.loop(0, NGRP)
        def _fire(j):
            pltpu.async_copy(
                tab_hbm.at[idx_v.at[pl.ds(j * GROUP, GROUP)]],
                rows_v.at[pl.ds(j * GROUP, GROUP)],
                sem,
            )

        pltpu.make_async_copy(tab_hbm.at[pl.ds(0, NROW)], rows_v, sem).wait()

        @pl.loop(0, M)
        def _extract(b):
            for ci in range(HC):
                p = b * HC + ci
                stage_v[b, pl.ds(ci * D, 16)] = rows_v[p, pl.ds(0, 16)]
                stage_v[b, pl.ds(ci * D + 16, 16)] = rows_v[p, pl.ds(16, 16)]

        pltpu.sync_copy(stage_v, emb_hbm.at[pl.ds(base, M)])


def _interleave_body(x_ref, e1_ref, e2_ref, out_ref):
    cont = x_ref[:, :NCONT].astype(jnp.float32)
    blk = jnp.concatenate([cont, e1_ref[...], e2_ref[...]], axis=1)
    out_ref[...] = blk.T


BM = 512


def _make_transpose(c0):
    return pl.pallas_call(
        functools.partial(_transpose_body, c0),
        out_shape=jax.ShapeDtypeStruct((HROWS, 128), jnp.float32),
        grid=(HC, TK),
        in_specs=[
            pl.BlockSpec(memory_space=pltpu.MemorySpace.HBM),
            pl.BlockSpec((NCAT, D, LAST), lambda c, k: (0, 0, 0)),
        ],
        out_specs=pl.BlockSpec((TV, 128), lambda c, k: (c * TK + k, 0)),
        scratch_shapes=[
            pltpu.VMEM((2, 128, TV), jnp.float32),
            pltpu.SemaphoreType.DMA((2, 4)),
        ],
    )


def _make_gather(c0):
    return pl.kernel(
        functools.partial(_gather_body, c0),
        out_type=jax.ShapeDtypeStruct((B, HC * D), jnp.float32),
        mesh=plsc.VectorSubcoreMesh(core_axis_name="c", subcore_axis_name="s"),
        compiler_params=pltpu.CompilerParams(use_tc_tiling_on_sc=False,
                                             needs_layout_passes=False),
        scratch_types=[
            pltpu.VMEM((M, XP), jnp.int32),
            pltpu.VMEM((NROW + 16,), jnp.int32),
            pltpu.VMEM((NROW, D), jnp.float32),
            pltpu.VMEM((M, HC * D), jnp.float32),
            pltpu.SemaphoreType.DMA,
        ],
    )


@jax.jit
def _run(x, tab_t):
    last = tab_t[:, :, V - LAST:V]
    xp = jnp.pad(x, ((0, 0), (0, XP - NCOLS)))

    t1 = _make_transpose(0)(tab_t, last)
    e1 = _make_gather(0)(xp, t1.reshape(4 * HROWS, D))
    t2 = _make_transpose(HC)(tab_t, last)
    e2 = _make_gather(HC)(xp, t2.reshape(4 * HROWS, D))

    out_t = pl.pallas_call(
        _interleave_body,
        out_shape=jax.ShapeDtypeStruct((OUT_W, B), jnp.float32),
        grid=(B // BM,),
        in_specs=[
            pl.BlockSpec((BM, NCOLS), lambda i: (i, 0)),
            pl.BlockSpec((BM, HC * D), lambda i: (i, 0)),
            pl.BlockSpec((BM, HC * D), lambda i: (i, 0)),
        ],
        out_specs=pl.BlockSpec((OUT_W, BM), lambda i: (0, i)),
    )(x, e1, e2)
    return out_t.T


def kernel(x, tables):
    return _run(x, tables.transpose(0, 2, 1))


# speedup vs baseline: 3.9254x; 1.0255x over previous
import functools

import jax
import jax.numpy as jnp
from jax import lax
from jax.experimental import pallas as pl
from jax.experimental.pallas import tpu as pltpu
from jax.experimental.pallas import tpu_sc as plsc

B = 16384
NCAT = 26
NCONT = 26
NCOLS = 52
V = 100000
D = 32
OUT_W = NCONT + NCAT * D
XP = 128
HC = NCAT // 2

NC = 2
NS = 16
NW = NC * NS
RW = B // NW
M = 64
NG = RW // M
NROW = M * HC
GROUP = 104
NGRP = NROW // GROUP

QA = 25088
TK = 7
TV = QA // TK
TAIL = 3200
LAST = 32
HROWS = HC * QA
TSTEPS = HC * TK


def _transpose_body(c0, tab_ref, last_ref, out_ref, buf, sems):
    c = pl.program_id(0)
    k = pl.program_id(1)
    i = c * TK + k
    slot = lax.rem(i, 2)
    nxt = lax.rem(i + 1, 2)

    def copy_descs(cc, kk, sl, tail):
        descs = []
        for r in range(3):
            descs.append(pltpu.make_async_copy(
                tab_ref.at[cc + c0, :, pl.ds(r * QA + kk * TV, TV)],
                buf.at[sl, pl.ds(r * D, D)], sems.at[sl, r]))
        if tail:
            descs.append(pltpu.make_async_copy(
                tab_ref.at[cc + c0, :, pl.ds(3 * QA + kk * TV, TAIL)],
                buf.at[sl, pl.ds(3 * D, D)].at[:, pl.ds(0, TAIL)],
                sems.at[sl, 3]))
        else:
            descs.append(pltpu.make_async_copy(
                tab_ref.at[cc + c0, :, pl.ds(3 * QA + kk * TV, TV)],
                buf.at[sl, pl.ds(3 * D, D)], sems.at[sl, 3]))
        return descs

    def start_copies(cc, kk, sl):
        @pl.when(kk < TK - 1)
        def _full():
            for d in copy_descs(cc, kk, sl, False):
                d.start()

        @pl.when(kk == TK - 1)
        def _tail():
            for d in copy_descs(cc, kk, sl, True):
                d.start()

    def wait_copies(cc, kk, sl):
        @pl.when(kk < TK - 1)
        def _full():
            for d in copy_descs(cc, kk, sl, False):
                d.wait()

        @pl.when(kk == TK - 1)
        def _tail():
            for d in copy_descs(cc, kk, sl, True):
                d.wait()

    @pl.when(i == 0)
    def _first():
        start_copies(c, k, slot)

    @pl.when(i < TSTEPS - 1)
    def _prefetch():
        start_copies(lax.div(i + 1, TK), lax.rem(i + 1, TK), nxt)

    wait_copies(c, k, slot)

    @pl.when(k == TK - 1)
    def _merge_tail():
        buf[slot, pl.ds(3 * D, D), pl.ds(TAIL, LAST)] = last_ref[c + c0]

    eye = (lax.broadcasted_iota(jnp.int32, (128, 128), 0)
           == lax.broadcasted_iota(jnp.int32, (128, 128), 1)
           ).astype(jnp.float32)
    out_ref[...] = lax.dot_general(
        buf[slot], eye,
        dimension_numbers=(((0,), (0,)), ((), ())),
        preferred_element_type=jnp.float32,
        precision=lax.Precision.HIGHEST)


def _gather_body(c0, x_hbm, tab_hbm, emb_hbm, x_v, idx_v, rows_v, stage_v,
                 sem):
    wid = lax.axis_index("s") * NC + lax.axis_index("c")
    iota = lax.iota(jnp.int32, 16)
    pat = iota * QA

    @pl.loop(0, NG)
    def _chunk(g):
        base = wid * RW + g * M

        pltpu.sync_copy(x_hbm.at[pl.ds(base, M)], x_v)

        @pl.loop(0, M)
        def _build(b):
            va = x_v[b, pl.ds(NCONT + c0, 16)]
            ra = ((va >= QA).astype(jnp.int32) + (va >= 2 * QA)
                  + (va >= 3 * QA))
            idx = (pat + va - ra * QA) * 4 + ra
            idx_v[pl.ds(b * HC, 16)] = jnp.where(iota < HC, idx, 0)

        @pl.loop(0, NGRP)
        def _fire(j):
            pltpu.async_copy(
                tab_hbm.at[idx_v.at[pl.ds(j * GROUP, GROUP)]],
                rows_v.at[pl.ds(j * GROUP, GROUP)],
                sem,
            )

        pltpu.make_async_copy(tab_hbm.at[pl.ds(0, NROW)], rows_v, sem).wait()

        @pl.loop(0, M)
        def _extract(b):
            for ci in range(HC):
                p = b * HC + ci
                stage_v[b, pl.ds(ci * D, 16)] = rows_v[p, pl.ds(0, 16)]
                stage_v[b, pl.ds(ci * D + 16, 16)] = rows_v[p, pl.ds(16, 16)]

        pltpu.sync_copy(stage_v, emb_hbm.at[pl.ds(base, M)])


def _interleave_body(x_ref, e1_ref, e2_ref, out_ref):
    cont = x_ref[:, :NCONT].astype(jnp.float32)
    blk = jnp.concatenate([cont, e1_ref[...], e2_ref[...]], axis=1)
    out_ref[...] = blk.T


BM = 512


def _make_transpose(c0):
    return pl.pallas_call(
        functools.partial(_transpose_body, c0),
        out_shape=jax.ShapeDtypeStruct((HROWS, 128), jnp.float32),
        grid=(HC, TK),
        in_specs=[
            pl.BlockSpec(memory_space=pltpu.MemorySpace.HBM),
            pl.BlockSpec((NCAT, D, LAST), lambda c, k: (0, 0, 0)),
        ],
        out_specs=pl.BlockSpec((TV, 128), lambda c, k: (c * TK + k, 0)),
        scratch_shapes=[
            pltpu.VMEM((2, 128, TV), jnp.float32),
            pltpu.SemaphoreType.DMA((2, 4)),
        ],
    )


def _make_gather(c0):
    return pl.kernel(
        functools.partial(_gather_body, c0),
        out_type=jax.ShapeDtypeStruct((B, HC * D), jnp.float32),
        mesh=plsc.VectorSubcoreMesh(core_axis_name="c", subcore_axis_name="s"),
        compiler_params=pltpu.CompilerParams(use_tc_tiling_on_sc=False,
                                             needs_layout_passes=False),
        scratch_types=[
            pltpu.VMEM((M, XP), jnp.int32),
            pltpu.VMEM((NROW + 16,), jnp.int32),
            pltpu.VMEM((NROW, D), jnp.float32),
            pltpu.VMEM((M, HC * D), jnp.float32),
            pltpu.SemaphoreType.DMA,
        ],
    )


@jax.jit
def _run(x, tab_t):
    last = tab_t[:, :, V - LAST:V]
    xp = jnp.pad(x, ((0, 0), (0, XP - NCOLS)))

    t1 = _make_transpose(0)(tab_t, last)
    e1 = _make_gather(0)(xp, t1.reshape(4 * HROWS, D))
    t2 = _make_transpose(HC)(tab_t, last)
    e2 = _make_gather(HC)(xp, t2.reshape(4 * HROWS, D))

    out_t = pl.pallas_call(
        _interleave_body,
        out_shape=jax.ShapeDtypeStruct((OUT_W, B), jnp.float32),
        grid=(B // BM,),
        in_specs=[
            pl.BlockSpec((BM, NCOLS), lambda i: (i, 0)),
            pl.BlockSpec((BM, HC * D), lambda i: (i, 0)),
            pl.BlockSpec((BM, HC * D), lambda i: (i, 0)),
        ],
        out_specs=pl.BlockSpec((OUT_W, BM), lambda i: (0, i)),
    )(x, e1, e2)
    return out_t.T


def kernel(x, tables):
    return _run(x, tables.transpose(0, 2, 1))
